# baseline (speedup 1.0000x reference)
import jax
import jax.numpy as jnp
from jax.experimental import pallas as pl
from jax.experimental.pallas import tpu as pltpu

_B = 2000


def _body(x_ref, t_ref, w_ref, o_ref):
    xb = x_ref[...].astype(jnp.bfloat16)
    tb = t_ref[...]
    D = x_ref.shape[1]
    T = (w_ref.shape[0] - 128) // D
    zero = jnp.zeros_like(xb)
    parts = [jnp.where(tb == t, xb, zero) for t in range(T)]
    oh = jnp.concatenate(
        [(tb == t).astype(jnp.bfloat16) for t in range(T)], axis=1)
    ohp = jnp.pad(oh, ((0, 0), (0, 128 - T)))
    xz = jnp.concatenate(parts + [ohp], axis=1)
    o_ref[...] = jnp.dot(xz, w_ref[...], preferred_element_type=jnp.float32)


def kernel(x, node_types, W, b):
    N, D = x.shape
    T, H, _ = W.shape
    assert N % _B == 0
    nt2 = node_types.astype(jnp.int32).reshape(N, 1)
    K = T * D + 128
    Wb = jnp.concatenate(
        [jnp.swapaxes(W, 1, 2).reshape(T * D, H),
         b,
         jnp.zeros((128 - T, H), W.dtype)],
        axis=0).astype(jnp.bfloat16)
    return pl.pallas_call(
        _body,
        grid=(N // _B,),
        in_specs=[
            pl.BlockSpec((_B, D), lambda i: (i, 0)),
            pl.BlockSpec((_B, 1), lambda i: (i, 0)),
            pl.BlockSpec((K, H), lambda i: (0, 0)),
        ],
        out_specs=pl.BlockSpec((_B, H), lambda i: (i, 0)),
        out_shape=jax.ShapeDtypeStruct((N, H), x.dtype),
        compiler_params=pltpu.CompilerParams(
            dimension_semantics=("parallel",),
        ),
    )(x, nt2, Wb)
